# baseline (speedup 1.0000x reference)
import jax
import jax.numpy as jnp
from jax.experimental import pallas as pl
from jax.experimental.pallas import tpu as pltpu

_BATCH = 4
_BS = 2048


def _bcast_kernel(pe_ref, out_ref):
    out_ref[...] = pe_ref[...][None]


def kernel(x, pe_weight):
    batch = x.shape[0]
    max_len, d_model = pe_weight.shape
    assert batch == _BATCH and max_len % _BS == 0
    grid = (max_len // _BS, batch)
    return pl.pallas_call(
        _bcast_kernel,
        grid=grid,
        in_specs=[pl.BlockSpec((_BS, d_model), lambda i, b: (i, 0))],
        out_specs=pl.BlockSpec((1, _BS, d_model), lambda i, b: (b, i, 0)),
        out_shape=jax.ShapeDtypeStruct((batch, max_len, d_model), pe_weight.dtype),
    )(pe_weight)


# speedup vs baseline: 1.1519x; 1.1519x over previous
import jax
import jax.numpy as jnp
from jax.experimental import pallas as pl
from jax.experimental.pallas import tpu as pltpu

_BATCH = 4
_BS = 1024


def _bcast_kernel(pe_ref, out_ref):
    blk = pe_ref[...]
    out_ref[...] = jnp.broadcast_to(blk[None], (_BATCH,) + blk.shape)


def kernel(x, pe_weight):
    batch = x.shape[0]
    max_len, d_model = pe_weight.shape
    assert batch == _BATCH and max_len % _BS == 0
    grid = (max_len // _BS,)
    return pl.pallas_call(
        _bcast_kernel,
        grid=grid,
        in_specs=[pl.BlockSpec((_BS, d_model), lambda i: (i, 0))],
        out_specs=pl.BlockSpec((batch, _BS, d_model), lambda i: (0, i, 0)),
        out_shape=jax.ShapeDtypeStruct((batch, max_len, d_model), pe_weight.dtype),
    )(pe_weight)
